# Initial kernel scaffold; baseline (speedup 1.0000x reference)
import jax
import jax.numpy as jnp
from jax.experimental import pallas as pl


def kernel(node_features, edge_features, edge_index, W_e, b_e, g_e, bt_e, W_n, b_n, g_n, bt_n):
    raise NotImplementedError("write your pallas kernel here")


# speedup vs baseline: 9.8716x; 9.8716x over previous
import functools

import jax
import jax.numpy as jnp
from jax import lax
from jax.experimental import pallas as pl
from jax.experimental.pallas import tpu as pltpu
from jax.experimental.pallas import tpu_sc as plsc

F32 = jnp.float32
EPS_LN = 1e-5
EPS_MEAN = 1e-10
HIGH = lax.Precision.HIGHEST

BATCH = 80
GRP = 5


def _pq_body(nf_ref, ws_ref, wt_ref, p_ref, q_ref):
    nf = nf_ref[...]
    p_ref[...] = jnp.dot(nf, ws_ref[...], preferred_element_type=F32,
                         precision=HIGH)
    q_ref[...] = jnp.dot(nf, wt_ref[...], preferred_element_type=F32,
                         precision=HIGH)


def _edge_body(ef_ref, g_ref, wbd_ref, am_ref, b_ref, gam_ref, bet_ref,
               out_ref):
    x = ef_ref[...]
    y = x + jnp.dot(x, wbd_ref[...], preferred_element_type=F32,
                    precision=HIGH) + b_ref[...] + g_ref[...]
    mu = jnp.dot(y, am_ref[...], preferred_element_type=F32, precision=HIGH)
    d = y - mu
    var = jnp.dot(d * d, am_ref[...], preferred_element_type=F32,
                  precision=HIGH)
    out_ref[...] = d * lax.rsqrt(var + EPS_LN) * gam_ref[...] + bet_ref[...]


def _node_body(nf_ref, s_ref, c_ref, wn1_ref, wn2_ref, bn_ref, gn_ref,
               btn_ref, out_ref):
    nf = nf_ref[...]
    s = jnp.sum(s_ref[...], axis=0)
    c = jnp.sum(c_ref[...], axis=0)
    mean = s / (c + EPS_MEAN)
    y = (nf
         + jnp.dot(nf, wn1_ref[...], preferred_element_type=F32,
                   precision=HIGH)
         + jnp.dot(mean, wn2_ref[...], preferred_element_type=F32,
                   precision=HIGH)
         + bn_ref[...])
    mu = jnp.mean(y, axis=-1, keepdims=True)
    d = y - mu
    var = jnp.mean(d * d, axis=-1, keepdims=True)
    out_ref[...] = d * lax.rsqrt(var + EPS_LN) * gn_ref[...] + btn_ref[...]


def _make_sc_gather(N, E, de, NC, NS):
    NW = NC * NS
    EW = E // NW
    NB = EW // BATCH
    NG = NB // GRP
    assert EW * NW == E and NB * BATCH == EW and NG * GRP == NB
    mesh = plsc.VectorSubcoreMesh(core_axis_name="c", subcore_axis_name="s")

    @functools.partial(
        pl.kernel,
        out_type=jax.ShapeDtypeStruct((E, de), F32),
        mesh=mesh,
        scratch_types=[
            pltpu.VMEM((GRP, BATCH), jnp.int32),
            pltpu.VMEM((GRP, BATCH), jnp.int32),
            pltpu.VMEM((GRP, BATCH, de), F32),
            pltpu.SemaphoreType.DMA,
            pltpu.SemaphoreType.DMA,
            pltpu.SemaphoreType.DMA,
        ],
        compiler_params=pltpu.CompilerParams(use_tc_tiling_on_sc=False),
    )
    def sc_gather(p_hbm, q_hbm, src_hbm, tgt_hbm, g_hbm,
                  sidx, tidx, rows, semi, semg, semo):
        wid = lax.axis_index("s") * NC + lax.axis_index("c")
        base = wid * EW

        def group(g, carry):
            gb = base + g * (GRP * BATCH)
            ds = []
            for b in range(GRP):
                off = gb + b * BATCH
                ds.append(pltpu.async_copy(src_hbm.at[pl.ds(off, BATCH)],
                                           sidx.at[b], semi))
                ds.append(pltpu.async_copy(tgt_hbm.at[pl.ds(off, BATCH)],
                                           tidx.at[b], semi))
            for d in ds:
                d.wait()
            ds = [pltpu.async_copy(q_hbm.at[tidx.at[b]], rows.at[b], semg)
                  for b in range(GRP)]
            for d in ds:
                d.wait()
            ds = [pltpu.async_copy(p_hbm.at[sidx.at[b]], rows.at[b], semg,
                                   add=True)
                  for b in range(GRP)]
            for d in ds:
                d.wait()
            ds = [pltpu.async_copy(rows.at[b],
                                   g_hbm.at[pl.ds(gb + b * BATCH, BATCH)],
                                   semo)
                  for b in range(GRP)]
            for d in ds:
                d.wait()
            return carry

        lax.fori_loop(0, NG, group, 0)

    return sc_gather


def _make_sc_scatter(N, E, de, NC, NS):
    NW = NC * NS
    EW = E // NW
    NB = EW // BATCH
    NG = NB // GRP
    NPS = N // NS
    assert NPS * NS == N
    mesh = plsc.VectorSubcoreMesh(core_axis_name="c", subcore_axis_name="s")

    @functools.partial(
        pl.kernel,
        out_type=(jax.ShapeDtypeStruct((NC * N, de), F32),
                  jax.ShapeDtypeStruct((NC * N, de), F32)),
        mesh=mesh,
        scratch_types=[
            pltpu.VMEM((GRP, BATCH), jnp.int32),
            pltpu.VMEM((GRP, BATCH), jnp.int32),
            pltpu.VMEM((GRP, BATCH, de), F32),
            pltpu.VMEM((BATCH, de), F32),
            pltpu.VMEM((NPS, de), F32),
            pltpu.VMEM_SHARED((N, de), F32),
            pltpu.VMEM_SHARED((N, de), F32),
            pltpu.SemaphoreType.DMA,
            pltpu.SemaphoreType.DMA,
        ],
        compiler_params=pltpu.CompilerParams(use_tc_tiling_on_sc=False),
    )
    def sc_scatter(ne_hbm, src_hbm, tgt_hbm, sums_hbm, cnts_hbm,
                   sidx, tidx, rows, ones_v, tmp_v, acc_s, cnt_s,
                   semi, sems):
        cid = lax.axis_index("c")
        sid = lax.axis_index("s")
        wid = sid * NC + cid
        base = wid * EW

        def zrow(r, carry):
            tmp_v[r, :] = jnp.zeros((de,), F32)
            return carry

        lax.fori_loop(0, NPS, zrow, 0)
        pltpu.sync_copy(tmp_v, acc_s.at[pl.ds(sid * NPS, NPS)])
        pltpu.sync_copy(tmp_v, cnt_s.at[pl.ds(sid * NPS, NPS)])

        def orow(r, carry):
            ones_v[r, :] = jnp.ones((de,), F32)
            return carry

        lax.fori_loop(0, BATCH, orow, 0)
        plsc.subcore_barrier()

        def group(g, carry):
            gb = base + g * (GRP * BATCH)
            ds = []
            for b in range(GRP):
                off = gb + b * BATCH
                ds.append(pltpu.async_copy(src_hbm.at[pl.ds(off, BATCH)],
                                           sidx.at[b], semi))
                ds.append(pltpu.async_copy(tgt_hbm.at[pl.ds(off, BATCH)],
                                           tidx.at[b], semi))
                ds.append(pltpu.async_copy(ne_hbm.at[pl.ds(off, BATCH)],
                                           rows.at[b], semi))
            for d in ds:
                d.wait()
            ds = []
            for b in range(GRP):
                ds.append(pltpu.async_copy(rows.at[b], acc_s.at[sidx.at[b]],
                                           sems, add=True))
                ds.append(pltpu.async_copy(rows.at[b], acc_s.at[tidx.at[b]],
                                           sems, add=True))
                ds.append(pltpu.async_copy(ones_v, cnt_s.at[sidx.at[b]],
                                           sems, add=True))
                ds.append(pltpu.async_copy(ones_v, cnt_s.at[tidx.at[b]],
                                           sems, add=True))
            for d in ds:
                d.wait()
            return carry

        lax.fori_loop(0, NG, group, 0)
        plsc.subcore_barrier()

        pltpu.sync_copy(acc_s.at[pl.ds(sid * NPS, NPS)], tmp_v)
        pltpu.sync_copy(tmp_v, sums_hbm.at[pl.ds(cid * N + sid * NPS, NPS)])
        pltpu.sync_copy(cnt_s.at[pl.ds(sid * NPS, NPS)], tmp_v)
        pltpu.sync_copy(tmp_v, cnts_hbm.at[pl.ds(cid * N + sid * NPS, NPS)])

    return sc_scatter


def kernel(node_features, edge_features, edge_index, W_e, b_e, g_e, bt_e,
           W_n, b_n, g_n, bt_n):
    N, dn = node_features.shape
    E, de = edge_features.shape
    info = plsc.get_sparse_core_info()
    NC, NS = info.num_cores, info.num_subcores

    src = edge_index[0]
    tgt = edge_index[1]
    W_self = W_e[:de]
    W_src = W_e[de:de + dn]
    W_tgt = W_e[de + dn:]

    BN = 2000
    pq = pl.pallas_call(
        _pq_body,
        grid=(N // BN,),
        in_specs=[
            pl.BlockSpec((BN, dn), lambda i: (i, 0)),
            pl.BlockSpec((dn, de), lambda i: (0, 0)),
            pl.BlockSpec((dn, de), lambda i: (0, 0)),
        ],
        out_specs=[
            pl.BlockSpec((BN, de), lambda i: (i, 0)),
            pl.BlockSpec((BN, de), lambda i: (i, 0)),
        ],
        out_shape=[
            jax.ShapeDtypeStruct((N, de), F32),
            jax.ShapeDtypeStruct((N, de), F32),
        ],
    )
    P, Q = pq(node_features, W_src, W_tgt)

    G = _make_sc_gather(N, E, de, NC, NS)(P, Q, src, tgt)

    R = 128 // de
    ef128 = edge_features.reshape(E // R, 128)
    g128 = G.reshape(E // R, 128)
    eye = jnp.eye(R, dtype=F32)
    wbd = jnp.kron(eye, W_self)
    amean = jnp.kron(eye, jnp.full((de, de), 1.0 / de, F32))
    b128 = jnp.tile(b_e, R)[None]
    gam128 = jnp.tile(g_e, R)[None]
    bet128 = jnp.tile(bt_e, R)[None]
    BE = 2000
    ne128 = pl.pallas_call(
        _edge_body,
        grid=((E // R) // BE,),
        in_specs=[
            pl.BlockSpec((BE, 128), lambda i: (i, 0)),
            pl.BlockSpec((BE, 128), lambda i: (i, 0)),
            pl.BlockSpec((128, 128), lambda i: (0, 0)),
            pl.BlockSpec((128, 128), lambda i: (0, 0)),
            pl.BlockSpec((1, 128), lambda i: (0, 0)),
            pl.BlockSpec((1, 128), lambda i: (0, 0)),
            pl.BlockSpec((1, 128), lambda i: (0, 0)),
        ],
        out_specs=pl.BlockSpec((BE, 128), lambda i: (i, 0)),
        out_shape=jax.ShapeDtypeStruct((E // R, 128), F32),
    )(ef128, g128, wbd, amean, b128, gam128, bet128)
    new_edges = ne128.reshape(E, de)

    sums, cnts = _make_sc_scatter(N, E, de, NC, NS)(new_edges, src, tgt)
    sums3 = sums.reshape(NC, N, de)
    cnts3 = cnts.reshape(NC, N, de)

    new_nodes = pl.pallas_call(
        _node_body,
        grid=(N // BN,),
        in_specs=[
            pl.BlockSpec((BN, dn), lambda i: (i, 0)),
            pl.BlockSpec((NC, BN, de), lambda i: (0, i, 0)),
            pl.BlockSpec((NC, BN, de), lambda i: (0, i, 0)),
            pl.BlockSpec((dn, dn), lambda i: (0, 0)),
            pl.BlockSpec((de, dn), lambda i: (0, 0)),
            pl.BlockSpec((1, dn), lambda i: (0, 0)),
            pl.BlockSpec((1, dn), lambda i: (0, 0)),
            pl.BlockSpec((1, dn), lambda i: (0, 0)),
        ],
        out_specs=pl.BlockSpec((BN, dn), lambda i: (i, 0)),
        out_shape=jax.ShapeDtypeStruct((N, dn), F32),
    )(node_features, sums3, cnts3, W_n[:dn], W_n[dn:], b_n[None], g_n[None],
      bt_n[None])

    return new_nodes, new_edges
